# baseline (speedup 1.0000x reference)
import functools

import jax
import jax.numpy as jnp
from jax import lax
from jax.experimental import pallas as pl
from jax.experimental.pallas import tpu as pltpu
from jax.experimental.pallas import tpu_sc as plsc

_NEG = float("-inf")


def _chunk_top(x, vi, beam):
    big = jnp.int32(2**30)
    cv, ci = [], []
    for k in range(beam):
        m = jnp.max(x)
        g = jnp.min(jnp.where(x == m, vi, big))
        cv.append(m)
        ci.append(g)
        if k + 1 < beam:
            x = jnp.where(vi == g, _NEG, x)
    return cv, ci


def _topk_body(lg_ref, probs_ref, idx_ref, *, vocab, beam, cl):
    cand_v, cand_i = [], []
    cms, css = [], []
    n_chunks = -(-vocab // cl)
    for j in range(n_chunks):
        off = j * cl
        size = min(cl, vocab - off)
        x = lg_ref[:, pl.ds(off, size)]
        vi = jax.lax.broadcasted_iota(jnp.int32, (1, size), 1) + off
        cv, ci = _chunk_top(x, vi, beam)
        cand_v.extend(cv)
        cand_i.extend(ci)
        cms.append(cv[0])
        css.append(jnp.sum(jnp.exp(x - cv[0])))

    big = jnp.int32(2**30)
    nc = len(cand_v)
    lane_c = jax.lax.broadcasted_iota(jnp.int32, (1, nc), 1)
    candv = jnp.zeros((1, nc), jnp.float32)
    candi = jnp.zeros((1, nc), jnp.int32)
    for k in range(nc):
        candv = jnp.where(lane_c == k, cand_v[k], candv)
        candi = jnp.where(lane_c == k, cand_i[k], candi)

    mg = cms[0]
    for c in cms[1:]:
        mg = jnp.maximum(mg, c)
    sg = css[0] * jnp.exp(cms[0] - mg)
    for c, s in zip(cms[1:], css[1:]):
        sg = sg + s * jnp.exp(c - mg)
    lse = jnp.log(sg) + mg

    lane_b = jax.lax.broadcasted_iota(jnp.int32, (1, beam), 1)
    pv = jnp.zeros((1, beam), jnp.float32)
    iv = jnp.zeros((1, beam), jnp.int32)
    for k in range(beam):
        m = jnp.max(candv)
        g = jnp.min(jnp.where(candv == m, candi, big))
        pv = jnp.where(lane_b == k, m - lse, pv)
        iv = jnp.where(lane_b == k, g, iv)
        candv = jnp.where(candi == g, _NEG, candv)
    probs_ref[...] = pv
    idx_ref[...] = iv


def _sc_copy_body(*refs, n_kv, beam, seq_half):
    kv_in = refs[:n_kv]
    kv_out = refs[n_kv:2 * n_kv]
    buf, in_sem, out_sem = refs[2 * n_kv:]

    c = lax.axis_index("c")
    s = lax.axis_index("s")
    soff = c * seq_half

    for j in range(n_kv):
        pltpu.async_copy(kv_in[j].at[0, s, pl.ds(soff, seq_half), :],
                         buf, in_sem).wait()
        started = [
            pltpu.async_copy(buf, kv_out[j].at[b, s, pl.ds(soff, seq_half), :],
                             out_sem)
            for b in range(beam)
        ]
        for cp in started:
            cp.wait()


def kernel(kv_0, kv_1, kv_2, kv_3, kv_4, kv_5, kv_6, kv_7, kv_8, kv_9,
           kv_10, kv_11, logits, save_id, beam_size):
    kvs = [kv_0, kv_1, kv_2, kv_3, kv_4, kv_5, kv_6, kv_7, kv_8, kv_9,
           kv_10, kv_11]
    n_kv = len(kvs)
    beam = save_id.shape[0]
    kv_shape = kvs[0].shape
    vocab = logits.shape[-1]
    seq_half = kv_shape[2] // 2

    mesh = plsc.VectorSubcoreMesh(core_axis_name="c", subcore_axis_name="s")
    sc_copy = functools.partial(
        pl.kernel,
        mesh=mesh,
        out_type=[jax.ShapeDtypeStruct((beam,) + kv_shape[1:],
                                       jnp.float32)] * n_kv,
        scratch_types=[pltpu.VMEM((seq_half, kv_shape[3]), jnp.float32),
                       pltpu.SemaphoreType.DMA,
                       pltpu.SemaphoreType.DMA],
    )(functools.partial(_sc_copy_body, n_kv=n_kv, beam=beam,
                        seq_half=seq_half))
    kv_outs = list(sc_copy(*kvs))

    cl = 83456
    topk = pl.pallas_call(
        functools.partial(_topk_body, vocab=vocab, beam=beam, cl=cl),
        in_specs=[pl.BlockSpec(memory_space=pltpu.MemorySpace.VMEM)],
        out_specs=[pl.BlockSpec(memory_space=pltpu.MemorySpace.VMEM),
                   pl.BlockSpec(memory_space=pltpu.MemorySpace.VMEM)],
        out_shape=[jax.ShapeDtypeStruct((1, beam), jnp.float32),
                   jax.ShapeDtypeStruct((1, beam), jnp.int32)],
    )
    probs, idx = topk(logits)

    idx_t = idx.reshape(beam, 1)
    save_id_out = jnp.concatenate([save_id, idx_t], axis=-1)
    probs_t = probs.reshape(beam, 1)
    bz = jnp.asarray(beam_size, jnp.int32) - jnp.int32(beam)
    max_idx = idx_t[0] + bz
    return (*kv_outs, idx_t, save_id_out, probs_t, max_idx)
